# Initial kernel scaffold; baseline (speedup 1.0000x reference)
import jax
import jax.numpy as jnp
from jax.experimental import pallas as pl


def kernel(xyz1, xyz2, points1, points2, W1, b1, g1, be1, W2, b2, g2, be2):
    raise NotImplementedError("write your pallas kernel here")


# speedup vs baseline: 12.1472x; 12.1472x over previous
import jax
import jax.numpy as jnp
from jax import lax
from jax.experimental import pallas as pl
from jax.experimental.pallas import tpu as pltpu
from jax.experimental.pallas import tpu_sc as plsc

_B, _N, _S, _D1, _D2 = 16, 4096, 1024, 128, 256
_C1, _C2 = 256, 128
_P = _B * _N

_NB1 = 256


def _knn_body(x1_ref, x2_ref, gidx_ref, w_ref):
    b = pl.program_id(0)
    x1 = x1_ref[0]
    x2 = x2_ref[0]
    a0 = x1[:, 0:1]
    a1 = x1[:, 1:2]
    a2 = x1[:, 2:3]
    r0 = x2[0:1, :]
    r1 = x2[1:2, :]
    r2 = x2[2:3, :]
    s1 = a0 * a0 + a1 * a1 + a2 * a2
    s2 = r0 * r0 + r1 * r1 + r2 * r2
    prod = lax.dot_general(x1, x2, (((1,), (0,)), ((), ())),
                           preferred_element_type=jnp.float32)
    d = (s1 + s2) - 2.0 * prod

    iota = lax.broadcasted_iota(jnp.int32, (_NB1, _S), 1)
    dists = []
    idxs = []
    dd = d
    for k in range(3):
        m = jnp.min(dd, axis=1, keepdims=True)
        am = jnp.min(jnp.where(dd == m, iota, _S), axis=1, keepdims=True)
        dists.append(m)
        idxs.append(am)
        if k < 2:
            dd = jnp.where(iota == am, jnp.float32(jnp.inf), dd)

    r0 = 1.0 / (dists[0] + 1e-8)
    r1 = 1.0 / (dists[1] + 1e-8)
    r2 = 1.0 / (dists[2] + 1e-8)
    norm = r0 + r1 + r2
    w_ref[0] = jnp.concatenate([r0 / norm, r1 / norm, r2 / norm], axis=1)
    gidx_ref[0] = jnp.concatenate(idxs, axis=1) + b * _S


def _knn(xyz1t, xyz2):
    return pl.pallas_call(
        _knn_body,
        grid=(_B, _N // _NB1),
        in_specs=[
            pl.BlockSpec((1, _NB1, 3), lambda b, n: (b, n, 0)),
            pl.BlockSpec((1, 3, _S), lambda b, n: (b, 0, 0)),
        ],
        out_specs=[
            pl.BlockSpec((1, _NB1, 3), lambda b, n: (b, n, 0)),
            pl.BlockSpec((1, _NB1, 3), lambda b, n: (b, n, 0)),
        ],
        out_shape=[
            jax.ShapeDtypeStruct((_B, _N, 3), jnp.int32),
            jax.ShapeDtypeStruct((_B, _N, 3), jnp.float32),
        ],
    )(xyz1t, xyz2)


_NC, _NS = 2, 16
_NW = _NC * _NS
_PTS_W = _P // _NW
_CP = 16
_ROWS = 3 * _CP
_CHUNKS = _PTS_W // _CP


def _sc_body(table_hbm, gidx_hbm, w_hbm, out_hbm, idx_v, w_v, rows_v, out_v, sem):
    cid = lax.axis_index("c")
    sid = lax.axis_index("s")
    wid = sid * _NC + cid
    base = wid * _PTS_W
    pltpu.sync_copy(gidx_hbm.at[pl.ds(base * 3, 3 * _PTS_W)], idx_v)
    pltpu.sync_copy(w_hbm.at[pl.ds(base * 3, 3 * _PTS_W)], w_v)

    def chunk(ch, carry):
        r0 = ch * _ROWS
        pltpu.async_copy(table_hbm.at[idx_v.at[pl.ds(r0, _ROWS)]], rows_v, sem).wait()
        wv = [w_v[pl.ds(r0 + 16 * t, 16)] for t in range(3)]
        for p in range(_CP):
            w0 = wv[(3 * p) // 16][(3 * p) % 16]
            w1 = wv[(3 * p + 1) // 16][(3 * p + 1) % 16]
            w2 = wv[(3 * p + 2) // 16][(3 * p + 2) % 16]
            for j in range(_D2 // 16):
                sl = pl.ds(j * 16, 16)
                out_v[p, sl] = (rows_v[3 * p, sl] * w0
                                + rows_v[3 * p + 1, sl] * w1
                                + rows_v[3 * p + 2, sl] * w2)
        pltpu.sync_copy(out_v, out_hbm.at[pl.ds(base + ch * _CP, _CP)])
        return carry

    lax.fori_loop(0, _CHUNKS, chunk, 0)


def _sc_interp(table, gidx_flat, w_flat):
    mesh = plsc.VectorSubcoreMesh(core_axis_name="c", subcore_axis_name="s",
                                  num_cores=_NC, num_subcores=_NS)
    kern = pl.kernel(
        _sc_body,
        out_type=jax.ShapeDtypeStruct((_P, _D2), jnp.float32),
        mesh=mesh,
        scratch_types=[
            pltpu.VMEM((3 * _PTS_W,), jnp.int32),
            pltpu.VMEM((3 * _PTS_W,), jnp.float32),
            pltpu.VMEM((_ROWS, _D2), jnp.float32),
            pltpu.VMEM((_CP, _D2), jnp.float32),
            pltpu.SemaphoreType.DMA,
        ],
    )
    return kern(table, gidx_flat, w_flat)


_NB2 = 512


def _mlp1_body(p1_ref, it_ref, w1t_ref, b1_ref, y_ref, st_ref):
    i = pl.program_id(0)
    x1 = p1_ref[...]
    x2 = it_ref[...]
    y = (lax.dot_general(x1, w1t_ref[0:_D1, :], (((1,), (0,)), ((), ())),
                         preferred_element_type=jnp.float32)
         + lax.dot_general(x2, w1t_ref[_D1:_D1 + _D2, :], (((1,), (0,)), ((), ())),
                           preferred_element_type=jnp.float32)
         + b1_ref[...])
    y_ref[...] = y

    @pl.when(i == 0)
    def _init():
        st_ref[...] = jnp.zeros_like(st_ref)

    st_ref[0:1, :] += jnp.sum(y, axis=0, keepdims=True)
    st_ref[1:2, :] += jnp.sum(y * y, axis=0, keepdims=True)


def _mlp1(p1t, interp, w1t, b1):
    return pl.pallas_call(
        _mlp1_body,
        grid=(_P // _NB2,),
        in_specs=[
            pl.BlockSpec((_NB2, _D1), lambda i: (i, 0)),
            pl.BlockSpec((_NB2, _D2), lambda i: (i, 0)),
            pl.BlockSpec((_D1 + _D2, _C1), lambda i: (0, 0)),
            pl.BlockSpec((1, _C1), lambda i: (0, 0)),
        ],
        out_specs=[
            pl.BlockSpec((_NB2, _C1), lambda i: (i, 0)),
            pl.BlockSpec((8, _C1), lambda i: (0, 0)),
        ],
        out_shape=[
            jax.ShapeDtypeStruct((_P, _C1), jnp.float32),
            jax.ShapeDtypeStruct((8, _C1), jnp.float32),
        ],
        compiler_params=pltpu.CompilerParams(
            dimension_semantics=("arbitrary",)),
    )(p1t, interp, w1t, b1)


_NB3 = 512


def _mlp2_body(y1_ref, a1_ref, c1_ref, w2t_ref, b2_ref, y_ref, st_ref):
    i = pl.program_id(0)
    h = jnp.maximum(y1_ref[...] * a1_ref[...] + c1_ref[...], 0.0)
    y = lax.dot_general(h, w2t_ref[...], (((1,), (0,)), ((), ())),
                        preferred_element_type=jnp.float32) + b2_ref[...]
    y_ref[...] = y

    @pl.when(i == 0)
    def _init():
        st_ref[...] = jnp.zeros_like(st_ref)

    st_ref[0:1, :] += jnp.sum(y, axis=0, keepdims=True)
    st_ref[1:2, :] += jnp.sum(y * y, axis=0, keepdims=True)


def _mlp2(y1, a1, c1, w2t, b2):
    return pl.pallas_call(
        _mlp2_body,
        grid=(_P // _NB3,),
        in_specs=[
            pl.BlockSpec((_NB3, _C1), lambda i: (i, 0)),
            pl.BlockSpec((1, _C1), lambda i: (0, 0)),
            pl.BlockSpec((1, _C1), lambda i: (0, 0)),
            pl.BlockSpec((_C1, _C2), lambda i: (0, 0)),
            pl.BlockSpec((1, _C2), lambda i: (0, 0)),
        ],
        out_specs=[
            pl.BlockSpec((_NB3, _C2), lambda i: (i, 0)),
            pl.BlockSpec((8, _C2), lambda i: (0, 0)),
        ],
        out_shape=[
            jax.ShapeDtypeStruct((_P, _C2), jnp.float32),
            jax.ShapeDtypeStruct((8, _C2), jnp.float32),
        ],
        compiler_params=pltpu.CompilerParams(
            dimension_semantics=("arbitrary",)),
    )(y1, a1, c1, w2t, b2)


_NB4 = 1024


def _finish_body(y2_ref, a2_ref, c2_ref, out_ref):
    z = jnp.maximum(y2_ref[0] * a2_ref[...] + c2_ref[...], 0.0)
    out_ref[0] = z.T


def _finish(y2r, a2, c2):
    return pl.pallas_call(
        _finish_body,
        grid=(_B, _N // _NB4),
        in_specs=[
            pl.BlockSpec((1, _NB4, _C2), lambda b, n: (b, n, 0)),
            pl.BlockSpec((1, _C2), lambda b, n: (0, 0)),
            pl.BlockSpec((1, _C2), lambda b, n: (0, 0)),
        ],
        out_specs=pl.BlockSpec((1, _C2, _NB4), lambda b, n: (b, 0, n)),
        out_shape=jax.ShapeDtypeStruct((_B, _C2, _N), jnp.float32),
    )(y2r, a2, c2)


def kernel(xyz1, xyz2, points1, points2, W1, b1, g1, be1, W2, b2, g2, be2):
    p1t = jnp.transpose(points1, (0, 2, 1)).reshape(_P, _D1)
    table = jnp.transpose(points2, (0, 2, 1)).reshape(_B * _S, _D2)
    w1t = W1.T
    w2t = W2.T

    gidx, w3 = _knn(jnp.transpose(xyz1, (0, 2, 1)), xyz2)
    interp = _sc_interp(table, gidx.reshape(_P * 3), w3.reshape(_P * 3))

    y1, st1 = _mlp1(p1t, interp, w1t, b1.reshape(1, _C1))
    mean1 = st1[0] / _P
    var1 = st1[1] / _P - mean1 * mean1
    a1 = g1 * lax.rsqrt(var1 + 1e-5)
    c1 = be1 - mean1 * a1

    y2, st2 = _mlp2(y1, a1.reshape(1, _C1), c1.reshape(1, _C1), w2t,
                    b2.reshape(1, _C2))
    mean2 = st2[0] / _P
    var2 = st2[1] / _P - mean2 * mean2
    a2 = g2 * lax.rsqrt(var2 + 1e-5)
    c2 = be2 - mean2 * a2

    return _finish(y2.reshape(_B, _N, _C2), a2.reshape(1, _C2),
                   c2.reshape(1, _C2))


# speedup vs baseline: 14.1335x; 1.1635x over previous
import jax
import jax.numpy as jnp
from jax import lax
from jax.experimental import pallas as pl
from jax.experimental.pallas import tpu as pltpu
from jax.experimental.pallas import tpu_sc as plsc

_B, _N, _S, _D1, _D2 = 16, 4096, 1024, 128, 256
_C1, _C2 = 256, 128
_P = _B * _N

_NB1 = 256


def _knn_body(x1_ref, x2_ref, gidx_ref, w_ref):
    b = pl.program_id(0)
    x1 = x1_ref[0]
    x2 = x2_ref[0]
    a0 = x1[:, 0:1]
    a1 = x1[:, 1:2]
    a2 = x1[:, 2:3]
    r0 = x2[0:1, :]
    r1 = x2[1:2, :]
    r2 = x2[2:3, :]
    s1 = a0 * a0 + a1 * a1 + a2 * a2
    s2 = r0 * r0 + r1 * r1 + r2 * r2
    prod = lax.dot_general(x1, x2, (((1,), (0,)), ((), ())),
                           preferred_element_type=jnp.float32)
    d = (s1 + s2) - 2.0 * prod

    iota = lax.broadcasted_iota(jnp.int32, (_NB1, _S), 1)
    dists = []
    idxs = []
    dd = d
    for k in range(3):
        m = jnp.min(dd, axis=1, keepdims=True)
        am = jnp.min(jnp.where(dd == m, iota, _S), axis=1, keepdims=True)
        dists.append(m)
        idxs.append(am)
        if k < 2:
            dd = jnp.where(iota == am, jnp.float32(jnp.inf), dd)

    r0 = 1.0 / (dists[0] + 1e-8)
    r1 = 1.0 / (dists[1] + 1e-8)
    r2 = 1.0 / (dists[2] + 1e-8)
    norm = r0 + r1 + r2
    w_ref[0] = jnp.concatenate([r0 / norm, r1 / norm, r2 / norm], axis=1)
    gidx_ref[0] = jnp.concatenate(idxs, axis=1) + b * _S


def _knn(xyz1t, xyz2):
    return pl.pallas_call(
        _knn_body,
        grid=(_B, _N // _NB1),
        in_specs=[
            pl.BlockSpec((1, _NB1, 3), lambda b, n: (b, n, 0)),
            pl.BlockSpec((1, 3, _S), lambda b, n: (b, 0, 0)),
        ],
        out_specs=[
            pl.BlockSpec((1, _NB1, 3), lambda b, n: (b, n, 0)),
            pl.BlockSpec((1, _NB1, 3), lambda b, n: (b, n, 0)),
        ],
        out_shape=[
            jax.ShapeDtypeStruct((_B, _N, 3), jnp.int32),
            jax.ShapeDtypeStruct((_B, _N, 3), jnp.float32),
        ],
    )(xyz1t, xyz2)


_NC, _NS = 2, 16
_NW = _NC * _NS
_PTS_W = _P // _NW
_CP = 16
_ROWS = 3 * _CP
_CHUNKS = _PTS_W // _CP


def _sc_body(table_hbm, gidx_hbm, w_hbm, out_hbm,
             idx_v, w_v, rows0, rows1, out0, out1,
             sin0, sin1, sout0, sout1):
    cid = lax.axis_index("c")
    sid = lax.axis_index("s")
    wid = sid * _NC + cid
    base = wid * _PTS_W
    pltpu.sync_copy(gidx_hbm.at[pl.ds(base * 3, 3 * _PTS_W)], idx_v)
    pltpu.sync_copy(w_hbm.at[pl.ds(base * 3, 3 * _PTS_W)], w_v)

    def gather_start(ch, rows, sem):
        pltpu.make_async_copy(
            table_hbm.at[idx_v.at[pl.ds(ch * _ROWS, _ROWS)]], rows, sem).start()

    def gather_wait(rows, sem):
        pltpu.make_async_copy(
            table_hbm.at[idx_v.at[pl.ds(0, _ROWS)]], rows, sem).wait()

    def out_start(ch, outb, sem):
        pltpu.make_async_copy(
            outb, out_hbm.at[pl.ds(base + ch * _CP, _CP)], sem).start()

    def out_wait(outb, sem):
        pltpu.make_async_copy(
            outb, out_hbm.at[pl.ds(base, _CP)], sem).wait()

    def compute(ch, rows, outb):
        r0 = ch * _ROWS
        wv = [w_v[pl.ds(r0 + 16 * t, 16)] for t in range(_ROWS // 16)]
        for p in range(_CP):
            w0 = wv[(3 * p) // 16][(3 * p) % 16]
            w1 = wv[(3 * p + 1) // 16][(3 * p + 1) % 16]
            w2 = wv[(3 * p + 2) // 16][(3 * p + 2) % 16]
            for j in range(_D2 // 16):
                sl = pl.ds(j * 16, 16)
                outb[p, sl] = (rows[3 * p, sl] * w0
                               + rows[3 * p + 1, sl] * w1
                               + rows[3 * p + 2, sl] * w2)

    gather_start(0, rows0, sin0)

    def outer(o, carry):
        ch0 = 2 * o
        ch1 = 2 * o + 1
        gather_start(ch1, rows1, sin1)
        gather_wait(rows0, sin0)

        @pl.when(o > 0)
        def _():
            out_wait(out0, sout0)
        compute(ch0, rows0, out0)
        out_start(ch0, out0, sout0)

        @pl.when(o < _CHUNKS // 2 - 1)
        def _():
            gather_start(ch0 + 2, rows0, sin0)
        gather_wait(rows1, sin1)

        @pl.when(o > 0)
        def _():
            out_wait(out1, sout1)
        compute(ch1, rows1, out1)
        out_start(ch1, out1, sout1)
        return carry

    lax.fori_loop(0, _CHUNKS // 2, outer, 0)
    out_wait(out0, sout0)
    out_wait(out1, sout1)


def _sc_interp(table, gidx_flat, w_flat):
    mesh = plsc.VectorSubcoreMesh(core_axis_name="c", subcore_axis_name="s",
                                  num_cores=_NC, num_subcores=_NS)
    kern = pl.kernel(
        _sc_body,
        out_type=jax.ShapeDtypeStruct((_P, _D2), jnp.float32),
        mesh=mesh,
        scratch_types=[
            pltpu.VMEM((3 * _PTS_W,), jnp.int32),
            pltpu.VMEM((3 * _PTS_W,), jnp.float32),
            pltpu.VMEM((_ROWS, _D2), jnp.float32),
            pltpu.VMEM((_ROWS, _D2), jnp.float32),
            pltpu.VMEM((_CP, _D2), jnp.float32),
            pltpu.VMEM((_CP, _D2), jnp.float32),
            pltpu.SemaphoreType.DMA,
            pltpu.SemaphoreType.DMA,
            pltpu.SemaphoreType.DMA,
            pltpu.SemaphoreType.DMA,
        ],
    )
    return kern(table, gidx_flat, w_flat)


_NB2 = 512


def _mlp1_body(p1_ref, it_ref, w1t_ref, b1_ref, y_ref, st_ref):
    i = pl.program_id(0)
    x1 = p1_ref[...]
    x2 = it_ref[...]
    y = (lax.dot_general(x1, w1t_ref[0:_D1, :], (((1,), (0,)), ((), ())),
                         preferred_element_type=jnp.float32)
         + lax.dot_general(x2, w1t_ref[_D1:_D1 + _D2, :], (((1,), (0,)), ((), ())),
                           preferred_element_type=jnp.float32)
         + b1_ref[...])
    y_ref[...] = y

    @pl.when(i == 0)
    def _init():
        st_ref[...] = jnp.zeros_like(st_ref)

    st_ref[0:1, :] += jnp.sum(y, axis=0, keepdims=True)
    st_ref[1:2, :] += jnp.sum(y * y, axis=0, keepdims=True)


def _mlp1(p1t, interp, w1t, b1):
    return pl.pallas_call(
        _mlp1_body,
        grid=(_P // _NB2,),
        in_specs=[
            pl.BlockSpec((_NB2, _D1), lambda i: (i, 0)),
            pl.BlockSpec((_NB2, _D2), lambda i: (i, 0)),
            pl.BlockSpec((_D1 + _D2, _C1), lambda i: (0, 0)),
            pl.BlockSpec((1, _C1), lambda i: (0, 0)),
        ],
        out_specs=[
            pl.BlockSpec((_NB2, _C1), lambda i: (i, 0)),
            pl.BlockSpec((8, _C1), lambda i: (0, 0)),
        ],
        out_shape=[
            jax.ShapeDtypeStruct((_P, _C1), jnp.float32),
            jax.ShapeDtypeStruct((8, _C1), jnp.float32),
        ],
        compiler_params=pltpu.CompilerParams(
            dimension_semantics=("arbitrary",)),
    )(p1t, interp, w1t, b1)


_NB3 = 512


def _mlp2_body(y1_ref, a1_ref, c1_ref, w2t_ref, b2_ref, y_ref, st_ref):
    i = pl.program_id(0)
    h = jnp.maximum(y1_ref[...] * a1_ref[...] + c1_ref[...], 0.0)
    y = lax.dot_general(h, w2t_ref[...], (((1,), (0,)), ((), ())),
                        preferred_element_type=jnp.float32) + b2_ref[...]
    y_ref[...] = y

    @pl.when(i == 0)
    def _init():
        st_ref[...] = jnp.zeros_like(st_ref)

    st_ref[0:1, :] += jnp.sum(y, axis=0, keepdims=True)
    st_ref[1:2, :] += jnp.sum(y * y, axis=0, keepdims=True)


def _mlp2(y1, a1, c1, w2t, b2):
    return pl.pallas_call(
        _mlp2_body,
        grid=(_P // _NB3,),
        in_specs=[
            pl.BlockSpec((_NB3, _C1), lambda i: (i, 0)),
            pl.BlockSpec((1, _C1), lambda i: (0, 0)),
            pl.BlockSpec((1, _C1), lambda i: (0, 0)),
            pl.BlockSpec((_C1, _C2), lambda i: (0, 0)),
            pl.BlockSpec((1, _C2), lambda i: (0, 0)),
        ],
        out_specs=[
            pl.BlockSpec((_NB3, _C2), lambda i: (i, 0)),
            pl.BlockSpec((8, _C2), lambda i: (0, 0)),
        ],
        out_shape=[
            jax.ShapeDtypeStruct((_P, _C2), jnp.float32),
            jax.ShapeDtypeStruct((8, _C2), jnp.float32),
        ],
        compiler_params=pltpu.CompilerParams(
            dimension_semantics=("arbitrary",)),
    )(y1, a1, c1, w2t, b2)


_NB4 = 1024


def _finish_body(y2_ref, a2_ref, c2_ref, out_ref):
    z = jnp.maximum(y2_ref[0] * a2_ref[...] + c2_ref[...], 0.0)
    out_ref[0] = z.T


def _finish(y2r, a2, c2):
    return pl.pallas_call(
        _finish_body,
        grid=(_B, _N // _NB4),
        in_specs=[
            pl.BlockSpec((1, _NB4, _C2), lambda b, n: (b, n, 0)),
            pl.BlockSpec((1, _C2), lambda b, n: (0, 0)),
            pl.BlockSpec((1, _C2), lambda b, n: (0, 0)),
        ],
        out_specs=pl.BlockSpec((1, _C2, _NB4), lambda b, n: (b, 0, n)),
        out_shape=jax.ShapeDtypeStruct((_B, _C2, _N), jnp.float32),
    )(y2r, a2, c2)


def kernel(xyz1, xyz2, points1, points2, W1, b1, g1, be1, W2, b2, g2, be2):
    p1t = jnp.transpose(points1, (0, 2, 1)).reshape(_P, _D1)
    table = jnp.transpose(points2, (0, 2, 1)).reshape(_B * _S, _D2)
    w1t = W1.T
    w2t = W2.T

    gidx, w3 = _knn(jnp.transpose(xyz1, (0, 2, 1)), xyz2)
    interp = _sc_interp(table, gidx.reshape(_P * 3), w3.reshape(_P * 3))

    y1, st1 = _mlp1(p1t, interp, w1t, b1.reshape(1, _C1))
    mean1 = st1[0] / _P
    var1 = st1[1] / _P - mean1 * mean1
    a1 = g1 * lax.rsqrt(var1 + 1e-5)
    c1 = be1 - mean1 * a1

    y2, st2 = _mlp2(y1, a1.reshape(1, _C1), c1.reshape(1, _C1), w2t,
                    b2.reshape(1, _C2))
    mean2 = st2[0] / _P
    var2 = st2[1] / _P - mean2 * mean2
    a2 = g2 * lax.rsqrt(var2 + 1e-5)
    c2 = be2 - mean2 * a2

    return _finish(y2.reshape(_B, _N, _C2), a2.reshape(1, _C2),
                   c2.reshape(1, _C2))
